# Initial kernel scaffold; baseline (speedup 1.0000x reference)
import jax
import jax.numpy as jnp
from jax.experimental import pallas as pl


def kernel(x, edge_index, edge_weight, W_in, b_in, W_out, b_out):
    raise NotImplementedError("write your pallas kernel here")


# speedup vs baseline: 1.6792x; 1.6792x over previous
import functools

import jax
import jax.numpy as jnp
from jax import lax
from jax.experimental import pallas as pl
from jax.experimental.pallas import tpu as pltpu
from jax.experimental.pallas import tpu_sc as plsc

N_NODES = 10000
NP = 10240
F = 128
FH = F // 2
ALPHA = 0.8
LAYERS = 10

NC = 2
NS = 16
NW = NC * NS
K = 128
C = 80
EDGES_PAD = NW * C * K

ROWS_PER_TILE = NP // NS
ZR = 128

BM = 1024


def _mm_in_body(x_ref, w_ref, b_ref, oa_ref, ob_ref):
    h = (
        jnp.dot(x_ref[...], w_ref[...], preferred_element_type=jnp.float32)
        + b_ref[...]
    )
    oa_ref[...] = h[:, :FH]
    ob_ref[...] = h[:, FH:]


def _matmul_in(x, W, b):
    return pl.pallas_call(
        _mm_in_body,
        grid=(NP // BM,),
        in_specs=[
            pl.BlockSpec((BM, F), lambda i: (i, 0)),
            pl.BlockSpec((F, F), lambda i: (0, 0)),
            pl.BlockSpec((1, F), lambda i: (0, 0)),
        ],
        out_specs=[
            pl.BlockSpec((BM, FH), lambda i: (i, 0)),
            pl.BlockSpec((BM, FH), lambda i: (i, 0)),
        ],
        out_shape=[
            jax.ShapeDtypeStruct((NP, FH), jnp.float32),
            jax.ShapeDtypeStruct((NP, FH), jnp.float32),
        ],
    )(x, W, b.reshape(1, F))


def _combine_body(pa_ref, pb_ref, h0a_ref, h0b_ref, oa_ref, ob_ref):
    oa_ref[...] = (
        ALPHA * (pa_ref[0] + pa_ref[1]) + (1.0 - ALPHA) * h0a_ref[...]
    )
    ob_ref[...] = (
        ALPHA * (pb_ref[0] + pb_ref[1]) + (1.0 - ALPHA) * h0b_ref[...]
    )


def _combine(pa, pb, h0a, h0b):
    return pl.pallas_call(
        _combine_body,
        grid=(NP // BM,),
        in_specs=[
            pl.BlockSpec((NC, BM, FH), lambda i: (0, i, 0)),
            pl.BlockSpec((NC, BM, FH), lambda i: (0, i, 0)),
            pl.BlockSpec((BM, FH), lambda i: (i, 0)),
            pl.BlockSpec((BM, FH), lambda i: (i, 0)),
        ],
        out_specs=[
            pl.BlockSpec((BM, FH), lambda i: (i, 0)),
            pl.BlockSpec((BM, FH), lambda i: (i, 0)),
        ],
        out_shape=[
            jax.ShapeDtypeStruct((NP, FH), jnp.float32),
            jax.ShapeDtypeStruct((NP, FH), jnp.float32),
        ],
    )(pa, pb, h0a, h0b)


def _combine_mm_body(pa_ref, pb_ref, h0a_ref, h0b_ref, w_ref, b_ref, o_ref):
    ha = ALPHA * (pa_ref[0] + pa_ref[1]) + (1.0 - ALPHA) * h0a_ref[...]
    hb = ALPHA * (pb_ref[0] + pb_ref[1]) + (1.0 - ALPHA) * h0b_ref[...]
    hcat = jnp.concatenate([ha, hb], axis=-1)
    o_ref[...] = (
        jnp.dot(hcat, w_ref[...], preferred_element_type=jnp.float32)
        + b_ref[...]
    )


def _combine_matmul(pa, pb, h0a, h0b, W, b):
    return pl.pallas_call(
        _combine_mm_body,
        grid=(NP // BM,),
        in_specs=[
            pl.BlockSpec((NC, BM, FH), lambda i: (0, i, 0)),
            pl.BlockSpec((NC, BM, FH), lambda i: (0, i, 0)),
            pl.BlockSpec((BM, FH), lambda i: (i, 0)),
            pl.BlockSpec((BM, FH), lambda i: (i, 0)),
            pl.BlockSpec((F, F), lambda i: (0, 0)),
            pl.BlockSpec((1, F), lambda i: (0, 0)),
        ],
        out_specs=pl.BlockSpec((BM, F), lambda i: (i, 0)),
        out_shape=jax.ShapeDtypeStruct((NP, F), jnp.float32),
    )(pa, pb, h0a, h0b, W, b.reshape(1, F))


def _propagate_kernel(h_hbm, sd_hbm, w_hbm, out_hbm,
                      src_v, dst_v, w_v, rows, zbuf, acc, sem):
    cid = lax.axis_index("c")
    sid = lax.axis_index("s")
    wid = sid * NC + cid

    pltpu.sync_copy(sd_hbm.at[0, wid], src_v)
    pltpu.sync_copy(sd_hbm.at[1, wid], dst_v)
    pltpu.sync_copy(w_hbm.at[wid], w_v)

    def zrow(r, _):
        for fj in range(FH // 16):
            zbuf[r, pl.ds(fj * 16, 16)] = jnp.zeros((16,), jnp.float32)
        return 0
    lax.fori_loop(0, ZR, zrow, 0)

    base = sid * ROWS_PER_TILE

    def zcopy(i, _):
        pltpu.sync_copy(zbuf, acc.at[pl.ds(base + i * ZR, ZR)])
        return 0
    lax.fori_loop(0, ROWS_PER_TILE // ZR, zcopy, 0)

    plsc.subcore_barrier()

    def chunk(j, _):
        pltpu.async_copy(h_hbm.at[src_v.at[j]], rows, sem).wait()

        def group(g, _):
            w16 = w_v[j, pl.ds(g * 16, 16)]
            for e in range(16):
                r = g * 16 + e
                wv = jnp.full((16,), w16[e], jnp.float32)
                for fj in range(FH // 16):
                    sl = pl.ds(fj * 16, 16)
                    rows[r, sl] = rows[r, sl] * wv
            return 0
        lax.fori_loop(0, K // 16, group, 0)

        pltpu.sync_copy(rows, acc.at[dst_v.at[j]], add=True)
        return 0
    lax.fori_loop(0, C, chunk, 0)

    plsc.subcore_barrier()

    pltpu.sync_copy(acc.at[pl.ds(base, ROWS_PER_TILE)],
                    out_hbm.at[cid, pl.ds(base, ROWS_PER_TILE)])


def _make_propagate():
    mesh = plsc.VectorSubcoreMesh(
        core_axis_name="c", subcore_axis_name="s",
        num_cores=NC, num_subcores=NS)
    return functools.partial(
        pl.kernel,
        out_type=jax.ShapeDtypeStruct((NC, NP, FH), jnp.float32),
        mesh=mesh,
        compiler_params=pltpu.CompilerParams(use_tc_tiling_on_sc=False),
        scratch_types=[
            pltpu.VMEM((C, K), jnp.int32),
            pltpu.VMEM((C, K), jnp.int32),
            pltpu.VMEM((C, K), jnp.float32),
            pltpu.VMEM((K, FH), jnp.float32),
            pltpu.VMEM((ZR, FH), jnp.float32),
            pltpu.VMEM_SHARED((NP, FH), jnp.float32),
            pltpu.SemaphoreType.DMA,
        ],
    )(_propagate_kernel)


def kernel(x, edge_index, edge_weight, W_in, b_in, W_out, b_out):
    src = edge_index[0].astype(jnp.int32)
    dst = edge_index[1].astype(jnp.int32)
    e = src.shape[0]
    pad = EDGES_PAD - e
    zpad = jnp.zeros((pad,), jnp.int32)
    sd = jnp.stack([jnp.concatenate([src, zpad]),
                    jnp.concatenate([dst, zpad])]).reshape(2, NW, C, K)
    w3 = jnp.concatenate(
        [edge_weight, jnp.zeros((pad,), jnp.float32)]).reshape(NW, C, K)
    sd, w3 = lax.optimization_barrier((sd, w3))

    n_cls = W_out.shape[1]
    W_out_p = jnp.zeros((F, F), jnp.float32).at[:, :n_cls].set(W_out)
    b_out_p = jnp.zeros((F,), jnp.float32).at[:n_cls].set(b_out)

    x_p = jnp.zeros((NP, F), jnp.float32).at[:N_NODES].set(x)
    h0a, h0b = _matmul_in(x_p, W_in, b_in)
    propagate = _make_propagate()

    ha, hb = h0a, h0b
    for _ in range(LAYERS - 1):
        pa = propagate(ha, sd, w3)
        pb = propagate(hb, sd, w3)
        ha, hb = _combine(pa, pb, h0a, h0b)
    pa = propagate(ha, sd, w3)
    pb = propagate(hb, sd, w3)
    out_p = _combine_matmul(pa, pb, h0a, h0b, W_out_p, b_out_p)
    return out_p[:N_NODES, :n_cls]


# speedup vs baseline: 2.5470x; 1.5168x over previous
import functools

import jax
import jax.numpy as jnp
from jax import lax
from jax.experimental import pallas as pl
from jax.experimental.pallas import tpu as pltpu
from jax.experimental.pallas import tpu_sc as plsc

N_NODES = 10000
NP = 10240
F = 128
FH = F // 2
ALPHA = 0.8
LAYERS = 10

NC = 2
NS = 16
NW = NC * NS
K = 128
C = 80
EDGES_PAD = NW * C * K

ROWS_PER_TILE = NP // NS
ZR = 128

BM = 1024


def _mm_in_body(x_ref, w_ref, b_ref, oa_ref, ob_ref):
    h = (
        jnp.dot(x_ref[...], w_ref[...], preferred_element_type=jnp.float32)
        + b_ref[...]
    )
    oa_ref[...] = h[:, :FH]
    ob_ref[...] = h[:, FH:]


def _matmul_in(x, W, b):
    return pl.pallas_call(
        _mm_in_body,
        grid=(NP // BM,),
        in_specs=[
            pl.BlockSpec((BM, F), lambda i: (i, 0)),
            pl.BlockSpec((F, F), lambda i: (0, 0)),
            pl.BlockSpec((1, F), lambda i: (0, 0)),
        ],
        out_specs=[
            pl.BlockSpec((BM, FH), lambda i: (i, 0)),
            pl.BlockSpec((BM, FH), lambda i: (i, 0)),
        ],
        out_shape=[
            jax.ShapeDtypeStruct((NP, FH), jnp.float32),
            jax.ShapeDtypeStruct((NP, FH), jnp.float32),
        ],
    )(x, W, b.reshape(1, F))


def _combine_body(pa_ref, pb_ref, h0a_ref, h0b_ref, oa_ref, ob_ref):
    oa_ref[...] = (
        ALPHA * (pa_ref[0] + pa_ref[1]) + (1.0 - ALPHA) * h0a_ref[...]
    )
    ob_ref[...] = (
        ALPHA * (pb_ref[0] + pb_ref[1]) + (1.0 - ALPHA) * h0b_ref[...]
    )


def _combine(pa, pb, h0a, h0b):
    return pl.pallas_call(
        _combine_body,
        grid=(NP // BM,),
        in_specs=[
            pl.BlockSpec((NC, BM, FH), lambda i: (0, i, 0)),
            pl.BlockSpec((NC, BM, FH), lambda i: (0, i, 0)),
            pl.BlockSpec((BM, FH), lambda i: (i, 0)),
            pl.BlockSpec((BM, FH), lambda i: (i, 0)),
        ],
        out_specs=[
            pl.BlockSpec((BM, FH), lambda i: (i, 0)),
            pl.BlockSpec((BM, FH), lambda i: (i, 0)),
        ],
        out_shape=[
            jax.ShapeDtypeStruct((NP, FH), jnp.float32),
            jax.ShapeDtypeStruct((NP, FH), jnp.float32),
        ],
    )(pa, pb, h0a, h0b)


def _combine_mm_body(pa_ref, pb_ref, h0a_ref, h0b_ref, w_ref, b_ref, o_ref):
    ha = ALPHA * (pa_ref[0] + pa_ref[1]) + (1.0 - ALPHA) * h0a_ref[...]
    hb = ALPHA * (pb_ref[0] + pb_ref[1]) + (1.0 - ALPHA) * h0b_ref[...]
    hcat = jnp.concatenate([ha, hb], axis=-1)
    o_ref[...] = (
        jnp.dot(hcat, w_ref[...], preferred_element_type=jnp.float32)
        + b_ref[...]
    )


def _combine_matmul(pa, pb, h0a, h0b, W, b):
    return pl.pallas_call(
        _combine_mm_body,
        grid=(NP // BM,),
        in_specs=[
            pl.BlockSpec((NC, BM, FH), lambda i: (0, i, 0)),
            pl.BlockSpec((NC, BM, FH), lambda i: (0, i, 0)),
            pl.BlockSpec((BM, FH), lambda i: (i, 0)),
            pl.BlockSpec((BM, FH), lambda i: (i, 0)),
            pl.BlockSpec((F, F), lambda i: (0, 0)),
            pl.BlockSpec((1, F), lambda i: (0, 0)),
        ],
        out_specs=pl.BlockSpec((BM, F), lambda i: (i, 0)),
        out_shape=jax.ShapeDtypeStruct((NP, F), jnp.float32),
    )(pa, pb, h0a, h0b, W, b.reshape(1, F))


def _propagate_kernel(h_hbm, sd_hbm, w_hbm, out_hbm,
                      src_v, dst_v, w_v, rows_a, rows_b, zbuf, acc,
                      gsem_a, gsem_b, ssem_a, ssem_b):
    cid = lax.axis_index("c")
    sid = lax.axis_index("s")
    wid = sid * NC + cid

    pltpu.sync_copy(sd_hbm.at[0, wid], src_v)
    pltpu.sync_copy(sd_hbm.at[1, wid], dst_v)
    pltpu.sync_copy(w_hbm.at[wid], w_v)

    pltpu.async_copy(h_hbm.at[src_v.at[0]], rows_a, gsem_a)

    def zrow(r, _):
        for fj in range(FH // 16):
            zbuf[r, pl.ds(fj * 16, 16)] = jnp.zeros((16,), jnp.float32)
        return 0
    lax.fori_loop(0, ZR, zrow, 0)

    base = sid * ROWS_PER_TILE

    def zcopy(i, _):
        pltpu.sync_copy(zbuf, acc.at[pl.ds(base + i * ZR, ZR)])
        return 0
    lax.fori_loop(0, ROWS_PER_TILE // ZR, zcopy, 0)

    plsc.subcore_barrier()

    def mult(rows, j):
        def group(g, _):
            w16 = w_v[j, pl.ds(g * 16, 16)]
            for e in range(16):
                r = g * 16 + e
                wv = jnp.full((16,), w16[e], jnp.float32)
                for fj in range(FH // 16):
                    sl = pl.ds(fj * 16, 16)
                    rows[r, sl] = rows[r, sl] * wv
            return 0
        lax.fori_loop(0, K // 16, group, 0)

    ni = C // 2

    def body(i, _):
        j0 = 2 * i
        j1 = 2 * i + 1
        pltpu.make_async_copy(h_hbm.at[src_v.at[j0]], rows_a, gsem_a).wait()

        @pl.when(i > 0)
        def _():
            pltpu.make_async_copy(
                rows_b, acc.at[dst_v.at[j1]], ssem_b).wait()

        pltpu.async_copy(h_hbm.at[src_v.at[j1]], rows_b, gsem_b)
        mult(rows_a, j0)
        pltpu.async_copy(rows_a, acc.at[dst_v.at[j0]], ssem_a, add=True)
        pltpu.make_async_copy(h_hbm.at[src_v.at[j1]], rows_b, gsem_b).wait()

        @pl.when(i < ni - 1)
        def _():
            pltpu.make_async_copy(
                rows_a, acc.at[dst_v.at[j0]], ssem_a).wait()
            pltpu.async_copy(h_hbm.at[src_v.at[j0 + 2]], rows_a, gsem_a)

        mult(rows_b, j1)
        pltpu.async_copy(rows_b, acc.at[dst_v.at[j1]], ssem_b, add=True)
        return 0
    lax.fori_loop(0, ni, body, 0)

    pltpu.make_async_copy(rows_a, acc.at[dst_v.at[C - 2]], ssem_a).wait()
    pltpu.make_async_copy(rows_b, acc.at[dst_v.at[C - 1]], ssem_b).wait()

    plsc.subcore_barrier()

    pltpu.sync_copy(acc.at[pl.ds(base, ROWS_PER_TILE)],
                    out_hbm.at[cid, pl.ds(base, ROWS_PER_TILE)])


def _make_propagate():
    mesh = plsc.VectorSubcoreMesh(
        core_axis_name="c", subcore_axis_name="s",
        num_cores=NC, num_subcores=NS)
    return functools.partial(
        pl.kernel,
        out_type=jax.ShapeDtypeStruct((NC, NP, FH), jnp.float32),
        mesh=mesh,
        compiler_params=pltpu.CompilerParams(use_tc_tiling_on_sc=False),
        scratch_types=[
            pltpu.VMEM((C, K), jnp.int32),
            pltpu.VMEM((C, K), jnp.int32),
            pltpu.VMEM((C, K), jnp.float32),
            pltpu.VMEM((K, FH), jnp.float32),
            pltpu.VMEM((K, FH), jnp.float32),
            pltpu.VMEM((ZR, FH), jnp.float32),
            pltpu.VMEM_SHARED((NP, FH), jnp.float32),
            pltpu.SemaphoreType.DMA,
            pltpu.SemaphoreType.DMA,
            pltpu.SemaphoreType.DMA,
            pltpu.SemaphoreType.DMA,
        ],
    )(_propagate_kernel)


def kernel(x, edge_index, edge_weight, W_in, b_in, W_out, b_out):
    src = edge_index[0].astype(jnp.int32)
    dst = edge_index[1].astype(jnp.int32)
    e = src.shape[0]
    pad = EDGES_PAD - e
    zpad = jnp.zeros((pad,), jnp.int32)
    sd = jnp.stack([jnp.concatenate([src, zpad]),
                    jnp.concatenate([dst, zpad])]).reshape(2, NW, C, K)
    w3 = jnp.concatenate(
        [edge_weight, jnp.zeros((pad,), jnp.float32)]).reshape(NW, C, K)
    sd, w3 = lax.optimization_barrier((sd, w3))

    n_cls = W_out.shape[1]
    W_out_p = jnp.zeros((F, F), jnp.float32).at[:, :n_cls].set(W_out)
    b_out_p = jnp.zeros((F,), jnp.float32).at[:n_cls].set(b_out)

    x_p = jnp.zeros((NP, F), jnp.float32).at[:N_NODES].set(x)
    h0a, h0b = _matmul_in(x_p, W_in, b_in)
    propagate = _make_propagate()

    ha, hb = h0a, h0b
    for _ in range(LAYERS - 1):
        pa = propagate(ha, sd, w3)
        pb = propagate(hb, sd, w3)
        ha, hb = _combine(pa, pb, h0a, h0b)
    pa = propagate(ha, sd, w3)
    pb = propagate(hb, sd, w3)
    out_p = _combine_matmul(pa, pb, h0a, h0b, W_out_p, b_out_p)
    return out_p[:N_NODES, :n_cls]


# speedup vs baseline: 6.8173x; 2.6765x over previous
import functools

import jax
import jax.numpy as jnp
from jax import lax
from jax.experimental import pallas as pl
from jax.experimental.pallas import tpu as pltpu
from jax.experimental.pallas import tpu_sc as plsc

N_NODES = 10000
NP = 10240
F = 128
FQ = F // 4
NQ = F // FQ
ALPHA = 0.8
LAYERS = 10

NC = 2
NS = 16
NW = NC * NS
K = 128
C = 80
EDGES_PAD = NW * C * K

ROWS_PER_TILE = NP // NS
ZR = 128

BM = 1024


def _mm_in_body(x_ref, w_ref, b_ref, o_ref):
    o_ref[...] = (
        jnp.dot(x_ref[...], w_ref[...], preferred_element_type=jnp.float32)
        + b_ref[...]
    )


def _matmul_in(x, W, b):
    return pl.pallas_call(
        _mm_in_body,
        grid=(NP // BM,),
        in_specs=[
            pl.BlockSpec((BM, F), lambda i: (i, 0)),
            pl.BlockSpec((F, F), lambda i: (0, 0)),
            pl.BlockSpec((1, F), lambda i: (0, 0)),
        ],
        out_specs=pl.BlockSpec((BM, F), lambda i: (i, 0)),
        out_shape=jax.ShapeDtypeStruct((NP, F), jnp.float32),
    )(x, W, b.reshape(1, F))


def _combine_body(p_ref, h0_ref, o_ref):
    o_ref[...] = (
        ALPHA * (p_ref[0] + p_ref[1]) + (1.0 - ALPHA) * h0_ref[...]
    )


def _combine(p, h0):
    return pl.pallas_call(
        _combine_body,
        grid=(NP // BM,),
        in_specs=[
            pl.BlockSpec((NC, BM, F), lambda i: (0, i, 0)),
            pl.BlockSpec((BM, F), lambda i: (i, 0)),
        ],
        out_specs=pl.BlockSpec((BM, F), lambda i: (i, 0)),
        out_shape=jax.ShapeDtypeStruct((NP, F), jnp.float32),
    )(p, h0)


def _combine_mm_body(p_ref, h0_ref, w_ref, b_ref, o_ref):
    h = ALPHA * (p_ref[0] + p_ref[1]) + (1.0 - ALPHA) * h0_ref[...]
    o_ref[...] = (
        jnp.dot(h, w_ref[...], preferred_element_type=jnp.float32)
        + b_ref[...]
    )


def _combine_matmul(p, h0, W, b):
    return pl.pallas_call(
        _combine_mm_body,
        grid=(NP // BM,),
        in_specs=[
            pl.BlockSpec((NC, BM, F), lambda i: (0, i, 0)),
            pl.BlockSpec((BM, F), lambda i: (i, 0)),
            pl.BlockSpec((F, F), lambda i: (0, 0)),
            pl.BlockSpec((1, F), lambda i: (0, 0)),
        ],
        out_specs=pl.BlockSpec((BM, F), lambda i: (i, 0)),
        out_shape=jax.ShapeDtypeStruct((NP, F), jnp.float32),
    )(p, h0, W, b.reshape(1, F))


def _propagate_kernel(h_hbm, sd_hbm, w_hbm, out_hbm,
                      src_v, dst_v, w_v, rows_a, rows_b, zbuf, hstage, acc,
                      gsem_a, gsem_b, ssem_a, ssem_b):
    cid = lax.axis_index("c")
    sid = lax.axis_index("s")
    wid = sid * NC + cid
    base = sid * ROWS_PER_TILE

    pltpu.sync_copy(sd_hbm.at[0, wid], src_v)
    pltpu.sync_copy(sd_hbm.at[1, wid], dst_v)
    pltpu.sync_copy(w_hbm.at[wid], w_v)

    def zrow(r, _):
        for fj in range(FQ // 16):
            zbuf[r, pl.ds(fj * 16, 16)] = jnp.zeros((16,), jnp.float32)
        return 0
    lax.fori_loop(0, ZR, zrow, 0)

    def mult(rows, j):
        def group(g, _):
            w16 = w_v[j, pl.ds(g * 16, 16)]
            for e in range(16):
                r = g * 16 + e
                wv = jnp.full((16,), w16[e], jnp.float32)
                for fj in range(FQ // 16):
                    sl = pl.ds(fj * 16, 16)
                    rows[r, sl] = rows[r, sl] * wv
            return 0
        lax.fori_loop(0, K // 16, group, 0)

    ni = C // 2

    for q in range(NQ):
        pltpu.sync_copy(
            h_hbm.at[pl.ds(base, ROWS_PER_TILE), pl.ds(q * FQ, FQ)],
            hstage.at[pl.ds(base, ROWS_PER_TILE)])
        for i in range(ROWS_PER_TILE // ZR):
            pltpu.sync_copy(zbuf, acc.at[pl.ds(base + i * ZR, ZR)])
        plsc.subcore_barrier()

        pltpu.async_copy(hstage.at[src_v.at[0]], rows_a, gsem_a)

        def body(i, _):
            j0 = 2 * i
            j1 = 2 * i + 1
            pltpu.make_async_copy(
                hstage.at[src_v.at[j0]], rows_a, gsem_a).wait()

            @pl.when(i > 0)
            def _():
                pltpu.make_async_copy(
                    rows_b, acc.at[dst_v.at[j1]], ssem_b).wait()

            pltpu.async_copy(hstage.at[src_v.at[j1]], rows_b, gsem_b)
            mult(rows_a, j0)
            pltpu.async_copy(rows_a, acc.at[dst_v.at[j0]], ssem_a, add=True)
            pltpu.make_async_copy(
                hstage.at[src_v.at[j1]], rows_b, gsem_b).wait()

            @pl.when(i < ni - 1)
            def _():
                pltpu.make_async_copy(
                    rows_a, acc.at[dst_v.at[j0]], ssem_a).wait()
                pltpu.async_copy(hstage.at[src_v.at[j0 + 2]], rows_a, gsem_a)

            mult(rows_b, j1)
            pltpu.async_copy(rows_b, acc.at[dst_v.at[j1]], ssem_b, add=True)
            return 0
        lax.fori_loop(0, ni, body, 0)

        pltpu.make_async_copy(rows_a, acc.at[dst_v.at[C - 2]], ssem_a).wait()
        pltpu.make_async_copy(rows_b, acc.at[dst_v.at[C - 1]], ssem_b).wait()
        plsc.subcore_barrier()

        pltpu.sync_copy(
            acc.at[pl.ds(base, ROWS_PER_TILE)],
            out_hbm.at[cid, pl.ds(base, ROWS_PER_TILE), pl.ds(q * FQ, FQ)])


def _make_propagate():
    mesh = plsc.VectorSubcoreMesh(
        core_axis_name="c", subcore_axis_name="s",
        num_cores=NC, num_subcores=NS)
    return functools.partial(
        pl.kernel,
        out_type=jax.ShapeDtypeStruct((NC, NP, F), jnp.float32),
        mesh=mesh,
        compiler_params=pltpu.CompilerParams(use_tc_tiling_on_sc=False),
        scratch_types=[
            pltpu.VMEM((C, K), jnp.int32),
            pltpu.VMEM((C, K), jnp.int32),
            pltpu.VMEM((C, K), jnp.float32),
            pltpu.VMEM((K, FQ), jnp.float32),
            pltpu.VMEM((K, FQ), jnp.float32),
            pltpu.VMEM((ZR, FQ), jnp.float32),
            pltpu.VMEM_SHARED((NP, FQ), jnp.float32),
            pltpu.VMEM_SHARED((NP, FQ), jnp.float32),
            pltpu.SemaphoreType.DMA,
            pltpu.SemaphoreType.DMA,
            pltpu.SemaphoreType.DMA,
            pltpu.SemaphoreType.DMA,
        ],
    )(_propagate_kernel)


def kernel(x, edge_index, edge_weight, W_in, b_in, W_out, b_out):
    src = edge_index[0].astype(jnp.int32)
    dst = edge_index[1].astype(jnp.int32)
    e = src.shape[0]
    pad = EDGES_PAD - e
    zpad = jnp.zeros((pad,), jnp.int32)
    sd = jnp.stack([jnp.concatenate([src, zpad]),
                    jnp.concatenate([dst, zpad])]).reshape(2, NW, C, K)
    w3 = jnp.concatenate(
        [edge_weight, jnp.zeros((pad,), jnp.float32)]).reshape(NW, C, K)
    sd, w3 = lax.optimization_barrier((sd, w3))

    n_cls = W_out.shape[1]
    W_out_p = jnp.zeros((F, F), jnp.float32).at[:, :n_cls].set(W_out)
    b_out_p = jnp.zeros((F,), jnp.float32).at[:n_cls].set(b_out)

    x_p = jnp.zeros((NP, F), jnp.float32).at[:N_NODES].set(x)
    h0 = _matmul_in(x_p, W_in, b_in)
    propagate = _make_propagate()

    h = h0
    for _ in range(LAYERS - 1):
        p = propagate(h, sd, w3)
        h = _combine(p, h0)
    p = propagate(h, sd, w3)
    out_p = _combine_matmul(p, h0, W_out_p, b_out_p)
    return out_p[:N_NODES, :n_cls]
